# baseline (speedup 1.0000x reference)
import functools

import jax
import jax.numpy as jnp
from jax import lax
from jax.experimental import pallas as pl
from jax.experimental.pallas import tpu as pltpu
from jax.experimental.pallas import tpu_sc as plsc

_NC = 2
_NS = 16
_NW = _NC * _NS
_CH = 80
_NB = 12
_LD = 6
_ZR = 128
_NP = 10240
_BR = 1000


@functools.lru_cache(maxsize=None)
def _make_prop(e, d, gather):
    ept = e // _NW
    nch = ept // _CH
    rpt = _NP // _NS
    nz = rpt // _ZR
    assert ept * _NW == e and nch * _CH == ept and nz * _ZR == rpt
    mesh = plsc.VectorSubcoreMesh(core_axis_name="c", subcore_axis_name="s")

    def _fill(ref, nrows, val):
        def row(r, _):
            for j in range(d // 16):
                ref[r, pl.ds(j * 16, 16)] = jnp.full((16,), val, jnp.float32)
            return 0
        lax.fori_loop(0, nrows, row, 0)

    def _prologue(dst_hbm, dstv, zbuf, acc):
        c = lax.axis_index("c")
        s = lax.axis_index("s")
        w = c * _NS + s
        _fill(zbuf, _ZR, 0.0)
        row0 = pl.multiple_of(s * rpt, _ZR)
        for k in range(nz):
            pltpu.sync_copy(zbuf, acc.at[pl.ds(row0 + k * _ZR, _ZR)])
        pltpu.sync_copy(dst_hbm.at[pl.ds(w * ept, ept)], dstv)
        return c, w, row0

    def _epilogue(out_hbm, acc, c, row0):
        plsc.subcore_barrier()
        pltpu.sync_copy(acc.at[pl.ds(row0, rpt)], out_hbm.at[c, pl.ds(row0, rpt)])

    if gather:
        nblk = (nch - _LD - (_LD + 1)) // _NB
        ep0 = _LD + _NB * nblk
        assert nblk >= 1 and _NB == 2 * _LD

        def body(g_hbm, src_hbm, dst_hbm, out_hbm, srcv, dstv, rows, zbuf, acc,
                 gsem, ssem):
            c, w, row0 = _prologue(dst_hbm, dstv, zbuf, acc)
            pltpu.sync_copy(src_hbm.at[pl.ds(w * ept, ept)], srcv)
            plsc.subcore_barrier()

            def idx(v, m):
                return v.at[pl.ds(m * _CH, _CH)]

            def g_start(m, k):
                pltpu.async_copy(g_hbm.at[idx(srcv, m)], rows.at[k], gsem.at[k])

            def g_wait(m, k):
                pltpu.make_async_copy(
                    g_hbm.at[idx(srcv, m)], rows.at[k], gsem.at[k]).wait()

            def s_start(m, k):
                pltpu.async_copy(
                    rows.at[k], acc.at[idx(dstv, m)], ssem.at[k], add=True)

            def s_wait(m, k):
                pltpu.make_async_copy(
                    rows.at[k], acc.at[idx(dstv, m)], ssem.at[k]).wait()

            for m in range(_LD):
                g_start(m, m)
            for m in range(_LD):
                g_wait(m, m)
                s_start(m, m)
                g_start(m + _LD, m + _LD)

            def blk(j, _):
                base = _NB * j + _LD
                for k8 in range(_NB):
                    m = base + k8
                    bb = (_LD + k8) % _NB
                    g_wait(m, bb)
                    s_start(m, bb)
                    s_wait(m - _LD, (bb + _LD) % _NB)
                    g_start(m + _LD, (bb + _LD) % _NB)
                return 0
            lax.fori_loop(0, nblk, blk, 0)

            for m in range(ep0, nch):
                bb = m % _NB
                g_wait(m, bb)
                s_start(m, bb)
                s_wait(m - _LD, (m - _LD) % _NB)
                if m + _LD < nch:
                    g_start(m + _LD, (m + _LD) % _NB)
            for m in range(max(ep0, nch - _LD), nch):
                s_wait(m, m % _NB)
            _epilogue(out_hbm, acc, c, row0)

        scratch = [
            pltpu.VMEM((ept,), jnp.int32),
            pltpu.VMEM((ept,), jnp.int32),
            pltpu.VMEM((_NB, _CH, d), jnp.float32),
            pltpu.VMEM((_ZR, d), jnp.float32),
            pltpu.VMEM_SHARED((_NP, d), jnp.float32),
            pltpu.SemaphoreType.DMA((_NB,)),
            pltpu.SemaphoreType.DMA((_NB,)),
        ]
    else:
        nq = nch // 5
        assert nq * 5 == nch

        def body(dst_hbm, out_hbm, dstv, rows, zbuf, acc, ssem):
            c, w, row0 = _prologue(dst_hbm, dstv, zbuf, acc)
            _fill(rows, _CH, 1.0)
            plsc.subcore_barrier()

            def idx(m):
                return dstv.at[pl.ds(m * _CH, _CH)]

            for k in range(5):
                pltpu.async_copy(rows, acc.at[idx(k)], ssem.at[k], add=True)

            def blk(j, _):
                for k in range(5):
                    pltpu.make_async_copy(
                        rows, acc.at[idx(5 * j + k)], ssem.at[k]).wait()
                    pltpu.async_copy(
                        rows, acc.at[idx(5 * (j + 1) + k)], ssem.at[k], add=True)
                return 0
            lax.fori_loop(0, nq - 1, blk, 0)
            for k in range(5):
                pltpu.make_async_copy(
                    rows, acc.at[idx(5 * (nq - 1) + k)], ssem.at[k]).wait()
            _epilogue(out_hbm, acc, c, row0)

        scratch = [
            pltpu.VMEM((ept,), jnp.int32),
            pltpu.VMEM((_CH, d), jnp.float32),
            pltpu.VMEM((_ZR, d), jnp.float32),
            pltpu.VMEM_SHARED((_NP, d), jnp.float32),
            pltpu.SemaphoreType.DMA((5,)),
        ]

    return pl.kernel(
        body,
        mesh=mesh,
        out_type=jax.ShapeDtypeStruct((_NC, _NP, d), jnp.float32),
        scratch_types=scratch,
        compiler_params=pltpu.CompilerParams(use_tc_tiling_on_sc=False),
    )


def _rb(d):
    return pl.BlockSpec((_BR, d), lambda i: (i, 0))


def _ab(d):
    return pl.BlockSpec((2, _BR, d), lambda i: (0, i, 0))


def _full(a, b):
    return pl.BlockSpec((a, b), lambda i: (0, 0))


def _bc(x, w):
    return jnp.broadcast_to(x, (x.shape[0], w))


def _mm1(x_ref, w1_ref, y1_o):
    y1_o[...] = jnp.dot(x_ref[...], w1_ref[...],
                        preferred_element_type=jnp.float32)


def _tc1(cnt2_ref, y1_ref, u1_o):
    cnt = cnt2_ref[0] + cnt2_ref[1]
    dinv = lax.rsqrt(cnt[:, 0:1] + 1.0)
    u1_o[...] = jnp.concatenate([y1_ref[...] * dinv, _bc(dinv, 64)], axis=1)


def _tc2(acc_ref, u1_ref, b1_ref, w2_ref, u2_o):
    u1 = u1_ref[...]
    g1 = u1[:, :64]
    dinv = u1[:, 64:65]
    h1 = jnp.maximum(dinv * (acc_ref[0] + acc_ref[1] + g1) + b1_ref[...], 0.0)
    y2 = jnp.dot(h1, w2_ref[...], preferred_element_type=jnp.float32)
    u2_o[...] = jnp.concatenate([y2 * dinv, _bc(dinv, 96)], axis=1)


def _tc3(acc_ref, u2_ref, b2_ref, u3_o, z_o):
    u2 = u2_ref[...]
    g2 = u2[:, :32]
    dinv = u2[:, 32:33]
    z = jnp.maximum(dinv * (acc_ref[0] + acc_ref[1] + g2) + b2_ref[...], 0.0)
    z_o[...] = z
    u3_o[...] = jnp.concatenate([z * dinv, _bc(dinv, 96)], axis=1)


def _tc4(acc_ref, u3_ref, w3_ref, b3_ref, u4_o):
    u3 = u3_ref[...]
    g3 = u3[:, :32]
    dinv = u3[:, 32:33]
    pz = dinv * (acc_ref[0] + acc_ref[1] + g3)
    d1 = jnp.maximum(
        jnp.dot(pz, w3_ref[...], preferred_element_type=jnp.float32) + b3_ref[...],
        0.0)
    u4_o[...] = jnp.concatenate([d1 * dinv, _bc(dinv, 64)], axis=1)


def _tc5(acc_ref, u4_ref, w4_ref, b4_ref, xr_o):
    u4 = u4_ref[...]
    g4 = u4[:, :64]
    dinv = u4[:, 64:65]
    pd = dinv * (acc_ref[0] + acc_ref[1] + g4)
    xr_o[...] = jnp.dot(pd, w4_ref[...], preferred_element_type=jnp.float32) + b4_ref[...]


def _sds(shape):
    return jax.ShapeDtypeStruct(shape, jnp.float32)


def _grid_call(body, in_specs, out_specs, out_shapes, n):
    return pl.pallas_call(
        body,
        grid=(n // _BR,),
        in_specs=in_specs,
        out_specs=out_specs,
        out_shape=out_shapes,
    )


def kernel(x, edge_index, W1, b1, W2, b2, W3, b3, W4, b4):
    n = x.shape[0]
    e = edge_index.shape[1]
    dd = x.shape[1]
    h2 = W1.shape[1]
    h1 = W2.shape[1]
    dst1 = edge_index[1]
    src1 = lax.optimization_barrier(edge_index)[0]

    p64 = _make_prop(e, h2, True)
    p32 = _make_prop(e, h1, True)

    cnt2 = _make_prop(e, 16, False)(dst1)
    y1 = _grid_call(_mm1, [_rb(dd), _full(dd, h2)], _rb(h2), _sds((n, h2)),
                    n)(x, W1)
    u1 = _grid_call(
        _tc1, [_ab(16), _rb(h2)], _rb(128), _sds((n, 128)),
        n)(cnt2, y1)

    acc = p64(u1[:, :h2], src1, dst1)
    u2 = _grid_call(
        _tc2, [_ab(h2), _rb(128), _full(1, h2), _full(h2, h1)],
        _rb(128), _sds((n, 128)), n)(acc, u1, b1.reshape(1, -1), W2)

    acc = p32(u2[:, :h1], src1, dst1)
    u3, z = _grid_call(
        _tc3, [_ab(h1), _rb(128), _full(1, h1)],
        [_rb(128), _rb(h1)], [_sds((n, 128)), _sds((n, h1))],
        n)(acc, u2, b2.reshape(1, -1))

    acc = p32(u3[:, :h1], src1, dst1)
    u4 = _grid_call(
        _tc4, [_ab(h1), _rb(128), _full(h1, h2), _full(1, h2)],
        _rb(128), _sds((n, 128)), n)(acc, u3, W3, b3.reshape(1, -1))

    acc = p64(u4[:, :h2], src1, dst1)
    x_recon = _grid_call(
        _tc5, [_ab(h2), _rb(128), _full(h2, dd), _full(1, dd)],
        _rb(dd), _sds((n, dd)), n)(acc, u4, W4, b4.reshape(1, -1))

    return (x_recon, z)


# speedup vs baseline: 1.0349x; 1.0349x over previous
import functools

import jax
import jax.numpy as jnp
from jax import lax
from jax.experimental import pallas as pl
from jax.experimental.pallas import tpu as pltpu
from jax.experimental.pallas import tpu_sc as plsc

_NC = 2
_NS = 16
_NW = _NC * _NS
_CH = 80
_NB = 12
_LD = 6
_ZR = 128
_NP = 10240
_BR = 1000


@functools.lru_cache(maxsize=None)
def _make_prop(e, d, gather):
    ept = e // _NW
    nch = ept // _CH
    rpt = _NP // _NS
    nz = rpt // _ZR
    assert ept * _NW == e and nch * _CH == ept and nz * _ZR == rpt
    mesh = plsc.VectorSubcoreMesh(core_axis_name="c", subcore_axis_name="s")

    def _fill(ref, nrows, val):
        def row(r, _):
            for j in range(d // 16):
                ref[r, pl.ds(j * 16, 16)] = jnp.full((16,), val, jnp.float32)
            return 0
        lax.fori_loop(0, nrows, row, 0)

    def _prologue(dst_hbm, dstv, zbuf, acc):
        c = lax.axis_index("c")
        s = lax.axis_index("s")
        w = c * _NS + s
        _fill(zbuf, _ZR, 0.0)
        row0 = pl.multiple_of(s * rpt, _ZR)
        for k in range(nz):
            pltpu.sync_copy(zbuf, acc.at[pl.ds(row0 + k * _ZR, _ZR)])
        pltpu.sync_copy(dst_hbm.at[pl.ds(w * ept, ept)], dstv)
        return c, w, row0

    def _epilogue(out_hbm, acc, c, row0):
        plsc.subcore_barrier()
        pltpu.sync_copy(acc.at[pl.ds(row0, rpt)], out_hbm.at[c, pl.ds(row0, rpt)])

    if gather:
        nblk = (nch - _LD - (_LD + 1)) // _NB
        ep0 = _LD + _NB * nblk
        assert nblk >= 1 and _NB == 2 * _LD

        def body(g_hbm, src_hbm, dst_hbm, out_hbm, srcv, dstv, rows, zbuf, acc,
                 gsem, ssem):
            c, w, row0 = _prologue(dst_hbm, dstv, zbuf, acc)
            pltpu.sync_copy(src_hbm.at[pl.ds(w * ept, ept)], srcv)
            plsc.subcore_barrier()

            def idx(v, m):
                return v.at[pl.ds(m * _CH, _CH)]

            def g_start(m, k):
                pltpu.async_copy(g_hbm.at[idx(srcv, m)], rows.at[k], gsem.at[k])

            def g_wait(m, k):
                pltpu.make_async_copy(
                    g_hbm.at[idx(srcv, m)], rows.at[k], gsem.at[k]).wait()

            def s_start(m, k):
                pltpu.async_copy(
                    rows.at[k], acc.at[idx(dstv, m)], ssem.at[k], add=True)

            def s_wait(m, k):
                pltpu.make_async_copy(
                    rows.at[k], acc.at[idx(dstv, m)], ssem.at[k]).wait()

            for m in range(_LD):
                g_start(m, m)
            for m in range(_LD):
                g_wait(m, m)
                s_start(m, m)
                g_start(m + _LD, m + _LD)

            def blk(j, _):
                base = _NB * j + _LD
                for k8 in range(_NB):
                    m = base + k8
                    bb = (_LD + k8) % _NB
                    g_wait(m, bb)
                    s_start(m, bb)
                    s_wait(m - _LD, (bb + _LD) % _NB)
                    g_start(m + _LD, (bb + _LD) % _NB)
                return 0
            lax.fori_loop(0, nblk, blk, 0)

            for m in range(ep0, nch):
                bb = m % _NB
                g_wait(m, bb)
                s_start(m, bb)
                s_wait(m - _LD, (m - _LD) % _NB)
                if m + _LD < nch:
                    g_start(m + _LD, (m + _LD) % _NB)
            for m in range(max(ep0, nch - _LD), nch):
                s_wait(m, m % _NB)
            _epilogue(out_hbm, acc, c, row0)

        scratch = [
            pltpu.VMEM((ept,), jnp.int32),
            pltpu.VMEM((ept,), jnp.int32),
            pltpu.VMEM((_NB, _CH, d), jnp.float32),
            pltpu.VMEM((_ZR, d), jnp.float32),
            pltpu.VMEM_SHARED((_NP, d), jnp.float32),
            pltpu.SemaphoreType.DMA((_NB,)),
            pltpu.SemaphoreType.DMA((_NB,)),
        ]
    else:
        nq = nch // 5
        assert nq * 5 == nch

        def body(dst_hbm, out_hbm, dstv, rows, zbuf, acc, ssem):
            c, w, row0 = _prologue(dst_hbm, dstv, zbuf, acc)
            _fill(rows, _CH, 1.0)
            plsc.subcore_barrier()

            def idx(m):
                return dstv.at[pl.ds(m * _CH, _CH)]

            for k in range(5):
                pltpu.async_copy(rows, acc.at[idx(k)], ssem.at[k], add=True)

            def blk(j, _):
                for k in range(5):
                    pltpu.make_async_copy(
                        rows, acc.at[idx(5 * j + k)], ssem.at[k]).wait()
                    pltpu.async_copy(
                        rows, acc.at[idx(5 * (j + 1) + k)], ssem.at[k], add=True)
                return 0
            lax.fori_loop(0, nq - 1, blk, 0)
            for k in range(5):
                pltpu.make_async_copy(
                    rows, acc.at[idx(5 * (nq - 1) + k)], ssem.at[k]).wait()
            _epilogue(out_hbm, acc, c, row0)

        scratch = [
            pltpu.VMEM((ept,), jnp.int32),
            pltpu.VMEM((_CH, d), jnp.float32),
            pltpu.VMEM((_ZR, d), jnp.float32),
            pltpu.VMEM_SHARED((_NP, d), jnp.float32),
            pltpu.SemaphoreType.DMA((5,)),
        ]

    return pl.kernel(
        body,
        mesh=mesh,
        out_type=jax.ShapeDtypeStruct((_NC, _NP, d), jnp.float32),
        scratch_types=scratch,
        compiler_params=pltpu.CompilerParams(use_tc_tiling_on_sc=False),
    )


def _rb(d):
    return pl.BlockSpec((_BR, d), lambda i: (i, 0))


def _ab(d):
    return pl.BlockSpec((2, _BR, d), lambda i: (0, i, 0))


def _full(a, b):
    return pl.BlockSpec((a, b), lambda i: (0, 0))


def _bc(x, w):
    return jnp.broadcast_to(x, (x.shape[0], w))


def _tc1(cnt2_ref, x_ref, w1_ref, u1_o):
    cnt = cnt2_ref[0] + cnt2_ref[1]
    dinv = lax.rsqrt(cnt[:, 0:1] + 1.0)
    y1 = jnp.dot(x_ref[...], w1_ref[...], preferred_element_type=jnp.float32)
    u1_o[...] = jnp.concatenate([y1 * dinv, _bc(dinv, 64)], axis=1)


def _tc2(acc_ref, u1_ref, b1_ref, w2_ref, u2_o):
    u1 = u1_ref[...]
    g1 = u1[:, :64]
    dinv = u1[:, 64:65]
    h1 = jnp.maximum(dinv * (acc_ref[0] + acc_ref[1] + g1) + b1_ref[...], 0.0)
    y2 = jnp.dot(h1, w2_ref[...], preferred_element_type=jnp.float32)
    u2_o[...] = jnp.concatenate([y2 * dinv, _bc(dinv, 96)], axis=1)


def _tc3(acc_ref, u2_ref, b2_ref, u3_o, z_o):
    u2 = u2_ref[...]
    g2 = u2[:, :32]
    dinv = u2[:, 32:33]
    z = jnp.maximum(dinv * (acc_ref[0] + acc_ref[1] + g2) + b2_ref[...], 0.0)
    z_o[...] = z
    u3_o[...] = jnp.concatenate([z * dinv, _bc(dinv, 96)], axis=1)


def _tc4(acc_ref, u3_ref, w3_ref, b3_ref, u4_o):
    u3 = u3_ref[...]
    g3 = u3[:, :32]
    dinv = u3[:, 32:33]
    pz = dinv * (acc_ref[0] + acc_ref[1] + g3)
    d1 = jnp.maximum(
        jnp.dot(pz, w3_ref[...], preferred_element_type=jnp.float32) + b3_ref[...],
        0.0)
    u4_o[...] = jnp.concatenate([d1 * dinv, _bc(dinv, 64)], axis=1)


def _tc5(acc_ref, u4_ref, w4_ref, b4_ref, xr_o):
    u4 = u4_ref[...]
    g4 = u4[:, :64]
    dinv = u4[:, 64:65]
    pd = dinv * (acc_ref[0] + acc_ref[1] + g4)
    xr_o[...] = jnp.dot(pd, w4_ref[...], preferred_element_type=jnp.float32) + b4_ref[...]


def _sds(shape):
    return jax.ShapeDtypeStruct(shape, jnp.float32)


def _grid_call(body, in_specs, out_specs, out_shapes, n):
    return pl.pallas_call(
        body,
        grid=(n // _BR,),
        in_specs=in_specs,
        out_specs=out_specs,
        out_shape=out_shapes,
    )


def kernel(x, edge_index, W1, b1, W2, b2, W3, b3, W4, b4):
    n = x.shape[0]
    e = edge_index.shape[1]
    dd = x.shape[1]
    h2 = W1.shape[1]
    h1 = W2.shape[1]
    src1 = edge_index[0]
    dst1 = edge_index[1]

    p64 = _make_prop(e, h2, True)
    p32 = _make_prop(e, h1, True)

    cnt2 = _make_prop(e, 16, False)(dst1)
    u1 = _grid_call(
        _tc1, [_ab(16), _rb(dd), _full(dd, h2)], _rb(128), _sds((n, 128)),
        n)(cnt2, x, W1)

    acc = p64(u1[:, :h2], src1, dst1)
    u2 = _grid_call(
        _tc2, [_ab(h2), _rb(128), _full(1, h2), _full(h2, h1)],
        _rb(128), _sds((n, 128)), n)(acc, u1, b1.reshape(1, -1), W2)

    acc = p32(u2[:, :h1], src1, dst1)
    u3, z = _grid_call(
        _tc3, [_ab(h1), _rb(128), _full(1, h1)],
        [_rb(128), _rb(h1)], [_sds((n, 128)), _sds((n, h1))],
        n)(acc, u2, b2.reshape(1, -1))

    acc = p32(u3[:, :h1], src1, dst1)
    u4 = _grid_call(
        _tc4, [_ab(h1), _rb(128), _full(h1, h2), _full(1, h2)],
        _rb(128), _sds((n, 128)), n)(acc, u3, W3, b3.reshape(1, -1))

    acc = p64(u4[:, :h2], src1, dst1)
    x_recon = _grid_call(
        _tc5, [_ab(h2), _rb(128), _full(h2, dd), _full(1, dd)],
        _rb(dd), _sds((n, dd)), n)(acc, u4, W4, b4.reshape(1, -1))

    return (x_recon, z)
